# baseline (speedup 1.0000x reference)
import jax
import jax.numpy as jnp
from jax import lax
from jax.experimental import pallas as pl
from jax.experimental.pallas import tpu as pltpu
from jax.experimental.pallas import tpu_sc as plsc

N = 10000
D = 128
E = 320000
CHUNK = 128
R = E // CHUNK
RP = 2560
NC, NS = 2, 16
NW = NC * NS
ROWS_A = RP // NS
ROWS_B = RP // NW
NP = 10240
NTILE = NP // NS


def _prep_body(beta_ref, feat_ref, ew_ref, nh_ref, bw_ref, m_ref):
    i = pl.program_id(0)
    f = feat_ref[...]
    ss = jnp.sum(f * f, axis=1, keepdims=True)
    nrm = jnp.maximum(jnp.sqrt(ss), 1e-12)
    nh_ref[...] = f / nrm
    bw = beta_ref[0, 0] * ew_ref[...]
    row = lax.broadcasted_iota(jnp.int32, bw.shape, 0) + i * (RP // 5)
    valid = row < R
    bw_ref[...] = jnp.where(valid, bw, -1e30)
    bmax = jnp.max(jnp.where(valid, bw, -jnp.inf))

    @pl.when(i == 0)
    def _():
        m_ref[...] = jnp.full((8, 128), -jnp.inf, jnp.float32)

    m_ref[...] = jnp.maximum(m_ref[...], bmax)


def _out_body(eps_ref, feat_ref, h0_ref, h1_ref, o_ref):
    o_ref[...] = (1.0 + eps_ref[0, 0]) * feat_ref[...] + h0_ref[0] + h1_ref[0]


def _scale_rows(gb, pbuf):
    @pl.loop(0, 8)
    def _(g):
        pv = pbuf[pl.ds(g * 16, 16)]
        for j in range(16):
            pr = pv[j]
            r = g * 16 + j
            for k in range(8):
                csl = pl.ds(k * 16, 16)
                gb[r, csl] = gb[r, csl] * pr


def _sc_body(nh_hbm, bw_hbm, m_hbm, src_hbm, dst_hbm, out_hbm,
             h_sh, den_sh, sidx, didx, bwb, aidx, xrow,
             gbuf, pbuf, dnb, cbuf, mbuf, sem_r, sem_g, sem_s):
    c = lax.axis_index("c")
    s = lax.axis_index("s")
    w = c * NS + s
    gb = [gbuf.at[0], gbuf.at[1]]

    @pl.loop(0, CHUNK)
    def _(r):
        for k in range(8):
            gbuf[0, r, pl.ds(k * 16, 16)] = jnp.zeros((16,), jnp.float32)

    for k in range(8):
        cbuf[pl.ds(k * 16, 16)] = jnp.full((16,), 1e-30, jnp.float32)

    pltpu.sync_copy(m_hbm, mbuf)
    mv = mbuf[pl.ds(0, 16)]

    base = s * NTILE
    with jax.named_scope("ph_init"):
        for k in range(5):
            pltpu.sync_copy(gb[0], h_sh.at[pl.ds(base + k * CHUNK, CHUNK)])
            pltpu.sync_copy(cbuf, den_sh.at[pl.ds(base + k * CHUNK, CHUNK)])
        plsc.subcore_barrier()

    with jax.named_scope("ph_a"):
        @pl.loop(0, ROWS_A // 4)
        def _(t):
            r0 = s * ROWS_A + t * 4
            loads = []
            for b in range(4):
                loads.append(pltpu.async_copy(src_hbm.at[r0 + b], aidx.at[b], sem_r))
                loads.append(pltpu.async_copy(bw_hbm.at[r0 + b], xrow.at[b], sem_r))
            for d in loads:
                d.wait()
            adds = []
            for b in range(4):
                for v in range(8):
                    sl = pl.ds(v * 16, 16)
                    xrow[b, sl] = jnp.exp(xrow[b, sl] - mv)
                adds.append(pltpu.async_copy(
                    xrow.at[b], den_sh.at[aidx.at[b]], sem_s, add=True))
            for d in adds:
                d.wait()

        plsc.subcore_barrier()

    row0 = w * ROWS_B

    def _b_gathers(ch, g, slot):
        pltpu.async_copy(nh_hbm.at[sidx.at[slot]], gb[g], sem_g)

    def _b_wait_gathers(g, slot):
        pltpu.make_async_copy(nh_hbm.at[sidx.at[slot]], gb[g], sem_g).wait()

    with jax.named_scope("ph_b"):
        pltpu.sync_copy(src_hbm.at[row0], sidx.at[0])
        pltpu.sync_copy(dst_hbm.at[row0], didx.at[0])
        pltpu.sync_copy(bw_hbm.at[row0], bwb.at[0])
        _b_gathers(0, 0, 0)

        @pl.loop(0, ROWS_B // 4)
        def _(t):
            for j in range(4):
                ch = t * 4 + j
                g = j % 2
                ng = (j + 1) % 2
                nslot = (j + 1) % 4
                not_last = ch + 1 < ROWS_B

                @pl.when(not_last)
                def _():
                    pltpu.async_copy(src_hbm.at[row0 + ch + 1], sidx.at[nslot], sem_r)
                    pltpu.async_copy(dst_hbm.at[row0 + ch + 1], didx.at[nslot], sem_r)
                    pltpu.async_copy(bw_hbm.at[row0 + ch + 1], bwb.at[nslot], sem_r)

                _b_wait_gathers(g, j)

                @pl.when(not_last)
                def _():
                    @pl.when(ch >= 1)
                    def _():
                        pltpu.make_async_copy(nh_hbm.at[pl.ds(0, CHUNK)], gb[ng], sem_s).wait()

                    pltpu.make_async_copy(src_hbm.at[row0 + ch + 1], sidx.at[nslot], sem_r).wait()
                    pltpu.make_async_copy(dst_hbm.at[row0 + ch + 1], didx.at[nslot], sem_r).wait()
                    pltpu.make_async_copy(bw_hbm.at[row0 + ch + 1], bwb.at[nslot], sem_r).wait()
                    _b_gathers(ch + 1, ng, nslot)

                pltpu.sync_copy(den_sh.at[sidx.at[j]], dnb.at[g])
                for v in range(8):
                    sl = pl.ds(v * 16, 16)
                    pbuf[sl] = jnp.exp(bwb[j, sl] - mv) / dnb[g, sl]

                _scale_rows(gb[g], pbuf)
                pltpu.async_copy(gb[g], h_sh.at[didx.at[j]], sem_s, add=True)

        pltpu.make_async_copy(nh_hbm.at[pl.ds(0, CHUNK)], gb[0], sem_s).wait()
        pltpu.make_async_copy(nh_hbm.at[pl.ds(0, CHUNK)], gb[1], sem_s).wait()
        plsc.subcore_barrier()

    with jax.named_scope("ph_c"):
        for k in range(5):
            r0 = base + k * CHUNK
            pltpu.sync_copy(h_sh.at[pl.ds(r0, CHUNK)], gb[0])
            pltpu.sync_copy(gb[0], out_hbm.at[c, pl.ds(r0, CHUNK)])


@jax.jit
def kernel(feat, edge_index, edge_weight, beta, eps):
    src = edge_index[0].astype(jnp.int32)
    dst = edge_index[1].astype(jnp.int32)
    ewp = jnp.pad(edge_weight.reshape(R, CHUNK), ((0, RP - R), (0, 0)))
    pad_idx = (jnp.arange((RP - R) * CHUNK, dtype=jnp.int32) % N).reshape(
        RP - R, CHUNK)
    srcp = jnp.concatenate([src.reshape(R, CHUNK), pad_idx])
    dstp = jnp.concatenate([dst.reshape(R, CHUNK), pad_idx])
    beta2 = beta.reshape(1, 1)
    eps2 = eps.reshape(1, 1)

    nh, bw, m = pl.pallas_call(
        _prep_body,
        grid=(5,),
        in_specs=[
            pl.BlockSpec(memory_space=pltpu.SMEM),
            pl.BlockSpec((2000, 128), lambda i: (i, 0)),
            pl.BlockSpec((512, 128), lambda i: (i, 0)),
        ],
        out_specs=[
            pl.BlockSpec((2000, 128), lambda i: (i, 0)),
            pl.BlockSpec((512, 128), lambda i: (i, 0)),
            pl.BlockSpec((8, 128), lambda i: (0, 0)),
        ],
        out_shape=[
            jax.ShapeDtypeStruct((N, D), jnp.float32),
            jax.ShapeDtypeStruct((RP, CHUNK), jnp.float32),
            jax.ShapeDtypeStruct((8, 128), jnp.float32),
        ],
    )(beta2, feat, ewp)

    sc_call = pl.kernel(
        _sc_body,
        out_type=jax.ShapeDtypeStruct((NC, NP, D), jnp.float32),
        mesh=plsc.VectorSubcoreMesh(core_axis_name="c", subcore_axis_name="s"),
        scratch_types=[
            pltpu.VMEM_SHARED((NP, D), jnp.float32),
            pltpu.VMEM_SHARED((NP,), jnp.float32),
            pltpu.VMEM((4, CHUNK), jnp.int32),
            pltpu.VMEM((4, CHUNK), jnp.int32),
            pltpu.VMEM((4, CHUNK), jnp.float32),
            pltpu.VMEM((4, CHUNK), jnp.int32),
            pltpu.VMEM((4, CHUNK), jnp.float32),
            pltpu.VMEM((2, CHUNK, CHUNK), jnp.float32),
            pltpu.VMEM((CHUNK,), jnp.float32),
            pltpu.VMEM((2, CHUNK), jnp.float32),
            pltpu.VMEM((CHUNK,), jnp.float32),
            pltpu.VMEM((CHUNK,), jnp.float32),
            pltpu.SemaphoreType.DMA,
            pltpu.SemaphoreType.DMA,
            pltpu.SemaphoreType.DMA,
        ],
    )
    hout = sc_call(nh, bw, m[0], srcp, dstp)

    out = pl.pallas_call(
        _out_body,
        grid=(5,),
        in_specs=[
            pl.BlockSpec(memory_space=pltpu.SMEM),
            pl.BlockSpec((2000, 128), lambda i: (i, 0)),
            pl.BlockSpec((1, 2000, 128), lambda i: (0, i, 0)),
            pl.BlockSpec((1, 2000, 128), lambda i: (1, i, 0)),
        ],
        out_specs=pl.BlockSpec((2000, 128), lambda i: (i, 0)),
        out_shape=jax.ShapeDtypeStruct((N, D), jnp.float32),
    )(eps2, feat, hout, hout)

    return out


# speedup vs baseline: 1.0316x; 1.0316x over previous
import jax
import jax.numpy as jnp
from jax import lax
from jax.experimental import pallas as pl
from jax.experimental.pallas import tpu as pltpu
from jax.experimental.pallas import tpu_sc as plsc

N = 10000
D = 128
E = 320000
CHUNK = 128
R = E // CHUNK
RP = 2560
NC, NS = 2, 16
NW = NC * NS
ROWS_A = RP // NS
ROWS_B = RP // NW
NP = 10240
NTILE = NP // NS


def _prep_body(beta_ref, feat_ref, ew_ref, nh_ref, bw_ref, m_ref):
    i = pl.program_id(0)
    f = feat_ref[...]
    ss = jnp.sum(f * f, axis=1, keepdims=True)
    nrm = jnp.maximum(jnp.sqrt(ss), 1e-12)
    nh_ref[...] = f / nrm
    bw = beta_ref[0, 0] * ew_ref[...]
    row = lax.broadcasted_iota(jnp.int32, bw.shape, 0) + i * (RP // 5)
    valid = row < R
    bw_ref[...] = jnp.where(valid, bw, -1e30)
    bmax = jnp.max(jnp.where(valid, bw, -jnp.inf))

    @pl.when(i == 0)
    def _():
        m_ref[...] = jnp.full((8, 128), -jnp.inf, jnp.float32)

    m_ref[...] = jnp.maximum(m_ref[...], bmax)


def _out_body(eps_ref, feat_ref, h0_ref, h1_ref, o_ref):
    o_ref[...] = (1.0 + eps_ref[0, 0]) * feat_ref[...] + h0_ref[0] + h1_ref[0]


def _scale_rows(gb, pbuf):
    @pl.loop(0, 8)
    def _(g):
        pv = pbuf[pl.ds(g * 16, 16)]
        for j in range(16):
            pr = pv[j]
            r = g * 16 + j
            for k in range(8):
                csl = pl.ds(k * 16, 16)
                gb[r, csl] = gb[r, csl] * pr


def _sc_body(nh_hbm, bw_hbm, m_hbm, src_hbm, dst_hbm, out_hbm,
             h_sh, den_sh, sidx, didx, bwb, aidx, xrow,
             gbuf, pbuf, dnb, cbuf, mbuf, sem_r, sem_g, sem_s, sem_d):
    c = lax.axis_index("c")
    s = lax.axis_index("s")
    w = c * NS + s
    gb = [gbuf.at[0], gbuf.at[1]]

    @pl.loop(0, CHUNK)
    def _(r):
        for k in range(8):
            gbuf[0, r, pl.ds(k * 16, 16)] = jnp.zeros((16,), jnp.float32)

    for k in range(8):
        cbuf[pl.ds(k * 16, 16)] = jnp.full((16,), 1e-30, jnp.float32)

    pltpu.sync_copy(m_hbm, mbuf)
    mv = mbuf[pl.ds(0, 16)]

    base = s * NTILE
    with jax.named_scope("ph_init"):
        for k in range(5):
            pltpu.sync_copy(gb[0], h_sh.at[pl.ds(base + k * CHUNK, CHUNK)])
            pltpu.sync_copy(cbuf, den_sh.at[pl.ds(base + k * CHUNK, CHUNK)])
        plsc.subcore_barrier()

    with jax.named_scope("ph_a"):
        @pl.loop(0, ROWS_A // 4)
        def _(t):
            r0 = s * ROWS_A + t * 4
            loads = []
            for b in range(4):
                loads.append(pltpu.async_copy(src_hbm.at[r0 + b], aidx.at[b], sem_r))
                loads.append(pltpu.async_copy(bw_hbm.at[r0 + b], xrow.at[b], sem_r))
            for d in loads:
                d.wait()
            adds = []
            for b in range(4):
                for v in range(8):
                    sl = pl.ds(v * 16, 16)
                    xrow[b, sl] = jnp.exp(xrow[b, sl] - mv)
                adds.append(pltpu.async_copy(
                    xrow.at[b], den_sh.at[aidx.at[b]], sem_s, add=True))
            for d in adds:
                d.wait()

        plsc.subcore_barrier()

    row0 = w * ROWS_B

    def _b_gathers(ch, g, slot):
        pltpu.async_copy(nh_hbm.at[sidx.at[slot]], gb[g], sem_g)
        pltpu.async_copy(den_sh.at[sidx.at[slot]], dnb.at[g], sem_d)

    def _b_wait_gathers(g, slot):
        pltpu.make_async_copy(nh_hbm.at[sidx.at[slot]], gb[g], sem_g).wait()
        pltpu.make_async_copy(den_sh.at[sidx.at[slot]], dnb.at[g], sem_d).wait()

    with jax.named_scope("ph_b"):
        pltpu.sync_copy(src_hbm.at[row0], sidx.at[0])
        pltpu.sync_copy(dst_hbm.at[row0], didx.at[0])
        pltpu.sync_copy(bw_hbm.at[row0], bwb.at[0])
        _b_gathers(0, 0, 0)

        @pl.loop(0, ROWS_B // 4)
        def _(t):
            for j in range(4):
                ch = t * 4 + j
                g = j % 2
                ng = (j + 1) % 2
                nslot = (j + 1) % 4
                not_last = ch + 1 < ROWS_B

                @pl.when(not_last)
                def _():
                    pltpu.async_copy(src_hbm.at[row0 + ch + 1], sidx.at[nslot], sem_r)
                    pltpu.async_copy(dst_hbm.at[row0 + ch + 1], didx.at[nslot], sem_r)
                    pltpu.async_copy(bw_hbm.at[row0 + ch + 1], bwb.at[nslot], sem_r)

                _b_wait_gathers(g, j)

                @pl.when(not_last)
                def _():
                    @pl.when(ch >= 1)
                    def _():
                        pltpu.make_async_copy(nh_hbm.at[pl.ds(0, CHUNK)], gb[ng], sem_s).wait()

                    pltpu.make_async_copy(src_hbm.at[row0 + ch + 1], sidx.at[nslot], sem_r).wait()
                    pltpu.make_async_copy(dst_hbm.at[row0 + ch + 1], didx.at[nslot], sem_r).wait()
                    pltpu.make_async_copy(bw_hbm.at[row0 + ch + 1], bwb.at[nslot], sem_r).wait()
                    _b_gathers(ch + 1, ng, nslot)

                for v in range(8):
                    sl = pl.ds(v * 16, 16)
                    pbuf[sl] = jnp.exp(bwb[j, sl] - mv) / dnb[g, sl]

                _scale_rows(gb[g], pbuf)
                pltpu.async_copy(gb[g], h_sh.at[didx.at[j]], sem_s, add=True)

        pltpu.make_async_copy(nh_hbm.at[pl.ds(0, CHUNK)], gb[0], sem_s).wait()
        pltpu.make_async_copy(nh_hbm.at[pl.ds(0, CHUNK)], gb[1], sem_s).wait()
        plsc.subcore_barrier()

    with jax.named_scope("ph_c"):
        for k in range(5):
            r0 = base + k * CHUNK
            pltpu.sync_copy(h_sh.at[pl.ds(r0, CHUNK)], gb[0])
            pltpu.sync_copy(gb[0], out_hbm.at[c, pl.ds(r0, CHUNK)])


@jax.jit
def kernel(feat, edge_index, edge_weight, beta, eps):
    src = edge_index[0].astype(jnp.int32)
    dst = edge_index[1].astype(jnp.int32)
    ewp = jnp.pad(edge_weight.reshape(R, CHUNK), ((0, RP - R), (0, 0)))
    pad_idx = (jnp.arange((RP - R) * CHUNK, dtype=jnp.int32) % N).reshape(
        RP - R, CHUNK)
    srcp = jnp.concatenate([src.reshape(R, CHUNK), pad_idx])
    dstp = jnp.concatenate([dst.reshape(R, CHUNK), pad_idx])
    beta2 = beta.reshape(1, 1)
    eps2 = eps.reshape(1, 1)

    nh, bw, m = pl.pallas_call(
        _prep_body,
        grid=(5,),
        in_specs=[
            pl.BlockSpec(memory_space=pltpu.SMEM),
            pl.BlockSpec((2000, 128), lambda i: (i, 0)),
            pl.BlockSpec((512, 128), lambda i: (i, 0)),
        ],
        out_specs=[
            pl.BlockSpec((2000, 128), lambda i: (i, 0)),
            pl.BlockSpec((512, 128), lambda i: (i, 0)),
            pl.BlockSpec((8, 128), lambda i: (0, 0)),
        ],
        out_shape=[
            jax.ShapeDtypeStruct((N, D), jnp.float32),
            jax.ShapeDtypeStruct((RP, CHUNK), jnp.float32),
            jax.ShapeDtypeStruct((8, 128), jnp.float32),
        ],
    )(beta2, feat, ewp)

    sc_call = pl.kernel(
        _sc_body,
        out_type=jax.ShapeDtypeStruct((NC, NP, D), jnp.float32),
        mesh=plsc.VectorSubcoreMesh(core_axis_name="c", subcore_axis_name="s"),
        scratch_types=[
            pltpu.VMEM_SHARED((NP, D), jnp.float32),
            pltpu.VMEM_SHARED((NP,), jnp.float32),
            pltpu.VMEM((4, CHUNK), jnp.int32),
            pltpu.VMEM((4, CHUNK), jnp.int32),
            pltpu.VMEM((4, CHUNK), jnp.float32),
            pltpu.VMEM((4, CHUNK), jnp.int32),
            pltpu.VMEM((4, CHUNK), jnp.float32),
            pltpu.VMEM((2, CHUNK, CHUNK), jnp.float32),
            pltpu.VMEM((CHUNK,), jnp.float32),
            pltpu.VMEM((2, CHUNK), jnp.float32),
            pltpu.VMEM((CHUNK,), jnp.float32),
            pltpu.VMEM((CHUNK,), jnp.float32),
            pltpu.SemaphoreType.DMA,
            pltpu.SemaphoreType.DMA,
            pltpu.SemaphoreType.DMA,
            pltpu.SemaphoreType.DMA,
        ],
    )
    hout = sc_call(nh, bw, m[0], srcp, dstp)

    out = pl.pallas_call(
        _out_body,
        grid=(5,),
        in_specs=[
            pl.BlockSpec(memory_space=pltpu.SMEM),
            pl.BlockSpec((2000, 128), lambda i: (i, 0)),
            pl.BlockSpec((1, 2000, 128), lambda i: (0, i, 0)),
            pl.BlockSpec((1, 2000, 128), lambda i: (1, i, 0)),
        ],
        out_specs=pl.BlockSpec((2000, 128), lambda i: (i, 0)),
        out_shape=jax.ShapeDtypeStruct((N, D), jnp.float32),
    )(eps2, feat, hout, hout)

    return out


# speedup vs baseline: 1.0888x; 1.0555x over previous
import jax
import jax.numpy as jnp
from jax import lax
from jax.experimental import pallas as pl
from jax.experimental.pallas import tpu as pltpu
from jax.experimental.pallas import tpu_sc as plsc

N = 10000
D = 128
E = 320000
CHUNK = 128
R = E // CHUNK
RP = 2560
NC, NS = 2, 16
NW = NC * NS
ROWS_A = RP // NS
ROWS_B = RP // NW
NP = 10240
NTILE = NP // NS


def _prep_body(beta_ref, feat_ref, ew_ref, nh_ref, bw_ref, m_ref):
    i = pl.program_id(0)
    f = feat_ref[...]
    ss = jnp.sum(f * f, axis=1, keepdims=True)
    nrm = jnp.maximum(jnp.sqrt(ss), 1e-12)
    nh_ref[...] = f / nrm
    bw = beta_ref[0, 0] * ew_ref[...]
    row = lax.broadcasted_iota(jnp.int32, bw.shape, 0) + i * (RP // 5)
    valid = row < R
    bw_ref[...] = jnp.where(valid, bw, -1e30)
    bmax = jnp.max(jnp.where(valid, bw, -jnp.inf))

    @pl.when(i == 0)
    def _():
        m_ref[...] = jnp.full((8, 128), -jnp.inf, jnp.float32)

    m_ref[...] = jnp.maximum(m_ref[...], bmax)


def _out_body(eps_ref, feat_ref, h0_ref, h1_ref, o_ref):
    o_ref[...] = (1.0 + eps_ref[0, 0]) * feat_ref[...] + h0_ref[0] + h1_ref[0]


def _scale_rows(gb, pbuf):
    @pl.loop(0, 8)
    def _(g):
        pv = pbuf[pl.ds(g * 16, 16)]
        for j in range(16):
            pr = pv[j]
            r = g * 16 + j
            for k in range(8):
                csl = pl.ds(k * 16, 16)
                gb[r, csl] = gb[r, csl] * pr


def _sc_body(nh_hbm, bw_hbm, m_hbm, src_hbm, dst_hbm, out_hbm,
             h_sh, den_sh, sidx, didx, bwb, aidx, xrow,
             gbuf, pbuf, dnb, cbuf, mbuf, sem_r, sem_g, sem_s, sem_d):
    c = lax.axis_index("c")
    s = lax.axis_index("s")
    w = c * NS + s
    gb = [gbuf.at[0], gbuf.at[1]]

    @pl.loop(0, CHUNK)
    def _(r):
        for k in range(8):
            gbuf[0, r, pl.ds(k * 16, 16)] = jnp.zeros((16,), jnp.float32)

    for k in range(8):
        cbuf[pl.ds(k * 16, 16)] = jnp.full((16,), 1e-30, jnp.float32)

    pltpu.sync_copy(m_hbm, mbuf)
    mv = mbuf[pl.ds(0, 16)]

    base = s * NTILE
    with jax.named_scope("ph_init"):
        for k in range(5):
            pltpu.sync_copy(gb[0], h_sh.at[pl.ds(base + k * CHUNK, CHUNK)])
            pltpu.sync_copy(cbuf, den_sh.at[pl.ds(base + k * CHUNK, CHUNK)])
        plsc.subcore_barrier()

    def _a_loads(t, p):
        r0 = s * ROWS_A + t * 4
        for b in range(4):
            pltpu.async_copy(src_hbm.at[r0 + b], aidx.at[p * 4 + b], sem_r)
            pltpu.async_copy(bw_hbm.at[r0 + b], xrow.at[p * 4 + b], sem_r)

    def _a_step(t, p):
        r0 = s * ROWS_A + t * 4
        for b in range(4):
            pltpu.make_async_copy(src_hbm.at[r0 + b], aidx.at[p * 4 + b], sem_r).wait()
            pltpu.make_async_copy(bw_hbm.at[r0 + b], xrow.at[p * 4 + b], sem_r).wait()

        @pl.when(t + 1 < ROWS_A // 4)
        def _():
            _a_loads(t + 1, 1 - p)

        adds = []
        for b in range(4):
            for v in range(8):
                sl = pl.ds(v * 16, 16)
                xrow[p * 4 + b, sl] = jnp.exp(xrow[p * 4 + b, sl] - mv)
            adds.append(pltpu.async_copy(
                xrow.at[p * 4 + b], den_sh.at[aidx.at[p * 4 + b]], sem_s, add=True))
        for d in adds:
            d.wait()

    with jax.named_scope("ph_a"):
        _a_loads(0, 0)

        @pl.loop(0, ROWS_A // 4, step=2)
        def _(t):
            _a_step(t, 0)
            _a_step(t + 1, 1)

        plsc.subcore_barrier()

    row0 = w * ROWS_B

    def _b_gathers(ch, g, slot):
        pltpu.async_copy(nh_hbm.at[sidx.at[slot]], gb[g], sem_g)
        pltpu.async_copy(den_sh.at[sidx.at[slot]], dnb.at[g], sem_d)

    def _b_wait_gathers(g, slot):
        pltpu.make_async_copy(nh_hbm.at[sidx.at[slot]], gb[g], sem_g).wait()
        pltpu.make_async_copy(den_sh.at[sidx.at[slot]], dnb.at[g], sem_d).wait()

    with jax.named_scope("ph_b"):
        pltpu.sync_copy(src_hbm.at[row0], sidx.at[0])
        pltpu.sync_copy(dst_hbm.at[row0], didx.at[0])
        pltpu.sync_copy(bw_hbm.at[row0], bwb.at[0])
        _b_gathers(0, 0, 0)
        pltpu.async_copy(src_hbm.at[row0 + 1], sidx.at[1], sem_r)
        pltpu.async_copy(dst_hbm.at[row0 + 1], didx.at[1], sem_r)
        pltpu.async_copy(bw_hbm.at[row0 + 1], bwb.at[1], sem_r)

        @pl.loop(0, ROWS_B // 4)
        def _(t):
            for j in range(4):
                ch = t * 4 + j
                g = j % 2
                ng = (j + 1) % 2
                nslot = (j + 1) % 4
                n2slot = (j + 2) % 4

                @pl.when(ch + 1 < ROWS_B)
                def _():
                    pltpu.make_async_copy(src_hbm.at[row0 + ch + 1], sidx.at[nslot], sem_r).wait()
                    pltpu.make_async_copy(dst_hbm.at[row0 + ch + 1], didx.at[nslot], sem_r).wait()
                    pltpu.make_async_copy(bw_hbm.at[row0 + ch + 1], bwb.at[nslot], sem_r).wait()

                    @pl.when(ch >= 1)
                    def _():
                        pltpu.make_async_copy(nh_hbm.at[pl.ds(0, CHUNK)], gb[ng], sem_s).wait()

                    _b_gathers(ch + 1, ng, nslot)

                @pl.when(ch + 2 < ROWS_B)
                def _():
                    pltpu.async_copy(src_hbm.at[row0 + ch + 2], sidx.at[n2slot], sem_r)
                    pltpu.async_copy(dst_hbm.at[row0 + ch + 2], didx.at[n2slot], sem_r)
                    pltpu.async_copy(bw_hbm.at[row0 + ch + 2], bwb.at[n2slot], sem_r)

                _b_wait_gathers(g, j)

                for v in range(8):
                    sl = pl.ds(v * 16, 16)
                    pbuf[sl] = jnp.exp(bwb[j, sl] - mv) / dnb[g, sl]

                _scale_rows(gb[g], pbuf)
                pltpu.async_copy(gb[g], h_sh.at[didx.at[j]], sem_s, add=True)

        pltpu.make_async_copy(nh_hbm.at[pl.ds(0, CHUNK)], gb[0], sem_s).wait()
        pltpu.make_async_copy(nh_hbm.at[pl.ds(0, CHUNK)], gb[1], sem_s).wait()
        plsc.subcore_barrier()

    with jax.named_scope("ph_c"):
        for k in range(5):
            r0 = base + k * CHUNK
            pltpu.sync_copy(h_sh.at[pl.ds(r0, CHUNK)], gb[0])
            pltpu.sync_copy(gb[0], out_hbm.at[c, pl.ds(r0, CHUNK)])


@jax.jit
def kernel(feat, edge_index, edge_weight, beta, eps):
    src = edge_index[0].astype(jnp.int32)
    dst = edge_index[1].astype(jnp.int32)
    ewp = jnp.pad(edge_weight.reshape(R, CHUNK), ((0, RP - R), (0, 0)))
    pad_idx = (jnp.arange((RP - R) * CHUNK, dtype=jnp.int32) % N).reshape(
        RP - R, CHUNK)
    srcp = jnp.concatenate([src.reshape(R, CHUNK), pad_idx])
    dstp = jnp.concatenate([dst.reshape(R, CHUNK), pad_idx])
    beta2 = beta.reshape(1, 1)
    eps2 = eps.reshape(1, 1)

    nh, bw, m = pl.pallas_call(
        _prep_body,
        grid=(5,),
        in_specs=[
            pl.BlockSpec(memory_space=pltpu.SMEM),
            pl.BlockSpec((2000, 128), lambda i: (i, 0)),
            pl.BlockSpec((512, 128), lambda i: (i, 0)),
        ],
        out_specs=[
            pl.BlockSpec((2000, 128), lambda i: (i, 0)),
            pl.BlockSpec((512, 128), lambda i: (i, 0)),
            pl.BlockSpec((8, 128), lambda i: (0, 0)),
        ],
        out_shape=[
            jax.ShapeDtypeStruct((N, D), jnp.float32),
            jax.ShapeDtypeStruct((RP, CHUNK), jnp.float32),
            jax.ShapeDtypeStruct((8, 128), jnp.float32),
        ],
    )(beta2, feat, ewp)

    sc_call = pl.kernel(
        _sc_body,
        out_type=jax.ShapeDtypeStruct((NC, NP, D), jnp.float32),
        mesh=plsc.VectorSubcoreMesh(core_axis_name="c", subcore_axis_name="s"),
        scratch_types=[
            pltpu.VMEM_SHARED((NP, D), jnp.float32),
            pltpu.VMEM_SHARED((NP,), jnp.float32),
            pltpu.VMEM((4, CHUNK), jnp.int32),
            pltpu.VMEM((4, CHUNK), jnp.int32),
            pltpu.VMEM((4, CHUNK), jnp.float32),
            pltpu.VMEM((8, CHUNK), jnp.int32),
            pltpu.VMEM((8, CHUNK), jnp.float32),
            pltpu.VMEM((2, CHUNK, CHUNK), jnp.float32),
            pltpu.VMEM((CHUNK,), jnp.float32),
            pltpu.VMEM((2, CHUNK), jnp.float32),
            pltpu.VMEM((CHUNK,), jnp.float32),
            pltpu.VMEM((CHUNK,), jnp.float32),
            pltpu.SemaphoreType.DMA,
            pltpu.SemaphoreType.DMA,
            pltpu.SemaphoreType.DMA,
            pltpu.SemaphoreType.DMA,
        ],
    )
    hout = sc_call(nh, bw, m[0], srcp, dstp)

    out = pl.pallas_call(
        _out_body,
        grid=(5,),
        in_specs=[
            pl.BlockSpec(memory_space=pltpu.SMEM),
            pl.BlockSpec((2000, 128), lambda i: (i, 0)),
            pl.BlockSpec((1, 2000, 128), lambda i: (0, i, 0)),
            pl.BlockSpec((1, 2000, 128), lambda i: (1, i, 0)),
        ],
        out_specs=pl.BlockSpec((2000, 128), lambda i: (i, 0)),
        out_shape=jax.ShapeDtypeStruct((N, D), jnp.float32),
    )(eps2, feat, hout, hout)

    return out


# speedup vs baseline: 1.1409x; 1.0478x over previous
import jax
import jax.numpy as jnp
from jax import lax
from jax.experimental import pallas as pl
from jax.experimental.pallas import tpu as pltpu
from jax.experimental.pallas import tpu_sc as plsc

N = 10000
D = 128
E = 320000
CHUNK = 128
R = E // CHUNK
RP = 2560
NC, NS = 2, 16
NW = NC * NS
ROWS_A = RP // NS
ROWS_B = RP // NW
NP = 10240
NTILE = NP // NS


def _prep_body(beta_ref, feat_ref, ew_ref, nh_ref, bw_ref, m_ref):
    i = pl.program_id(0)
    f = feat_ref[...]
    ss = jnp.sum(f * f, axis=1, keepdims=True)
    nrm = jnp.maximum(jnp.sqrt(ss), 1e-12)
    nh_ref[...] = f / nrm
    bw = beta_ref[0, 0] * ew_ref[...]
    row = lax.broadcasted_iota(jnp.int32, bw.shape, 0) + i * (RP // 5)
    valid = row < R
    bw_ref[...] = jnp.where(valid, bw, -1e30)
    bmax = jnp.max(jnp.where(valid, bw, -jnp.inf))

    @pl.when(i == 0)
    def _():
        m_ref[...] = jnp.full((8, 128), -jnp.inf, jnp.float32)

    m_ref[...] = jnp.maximum(m_ref[...], bmax)


def _out_body(eps_ref, feat_ref, h0_ref, h1_ref, o_ref):
    o_ref[...] = (1.0 + eps_ref[0, 0]) * feat_ref[...] + h0_ref[0] + h1_ref[0]


def _scale_rows(gb, pbuf):
    @pl.loop(0, 8)
    def _(g):
        pv = pbuf[pl.ds(g * 16, 16)]
        for j in range(16):
            pr = pv[j]
            r = g * 16 + j
            for k in range(8):
                csl = pl.ds(k * 16, 16)
                gb[r, csl] = gb[r, csl] * pr


def _sc_body(nh_hbm, bw_hbm, m_hbm, ei_hbm, out_hbm,
             h_sh, den_sh, sidx, didx, bwb, aidx, xrow,
             gbuf, pbuf, dnb, cbuf, mbuf, sem_r, sem_g, sem_s, sem_d):
    c = lax.axis_index("c")
    s = lax.axis_index("s")
    w = c * NS + s
    gb = [gbuf.at[0], gbuf.at[1]]

    @pl.loop(0, CHUNK)
    def _(r):
        for k in range(8):
            gbuf[0, r, pl.ds(k * 16, 16)] = jnp.zeros((16,), jnp.float32)

    for k in range(8):
        cbuf[pl.ds(k * 16, 16)] = jnp.full((16,), 1e-30, jnp.float32)

    pltpu.sync_copy(m_hbm, mbuf)
    mv = mbuf[pl.ds(0, 16)]

    base = s * NTILE
    with jax.named_scope("ph_init"):
        for k in range(5):
            pltpu.sync_copy(gb[0], h_sh.at[pl.ds(base + k * CHUNK, CHUNK)])
            pltpu.sync_copy(cbuf, den_sh.at[pl.ds(base + k * CHUNK, CHUNK)])
        plsc.subcore_barrier()

    def _a_loads(t, p):
        r0 = s * ROWS_A + t * 4
        for b in range(4):
            pltpu.async_copy(ei_hbm.at[0, r0 + b], aidx.at[p * 4 + b], sem_r)
            pltpu.async_copy(bw_hbm.at[r0 + b], xrow.at[p * 4 + b], sem_r)

    def _a_step(t, p):
        r0 = s * ROWS_A + t * 4
        for b in range(4):
            pltpu.make_async_copy(ei_hbm.at[0, r0 + b], aidx.at[p * 4 + b], sem_r).wait()
            pltpu.make_async_copy(bw_hbm.at[r0 + b], xrow.at[p * 4 + b], sem_r).wait()

        @pl.when(t + 1 < ROWS_A // 4)
        def _():
            _a_loads(t + 1, 1 - p)

        adds = []
        for b in range(4):
            for v in range(8):
                sl = pl.ds(v * 16, 16)
                xrow[p * 4 + b, sl] = jnp.exp(xrow[p * 4 + b, sl] - mv)
            adds.append(pltpu.async_copy(
                xrow.at[p * 4 + b], den_sh.at[aidx.at[p * 4 + b]], sem_s, add=True))
        for d in adds:
            d.wait()

    with jax.named_scope("ph_a"):
        _a_loads(0, 0)

        @pl.loop(0, ROWS_A // 4, step=2)
        def _(t):
            _a_step(t, 0)
            _a_step(t + 1, 1)

        plsc.subcore_barrier()

    row0 = w * ROWS_B

    def _b_gathers(ch, g, slot):
        pltpu.async_copy(nh_hbm.at[sidx.at[slot]], gb[g], sem_g)
        pltpu.async_copy(den_sh.at[sidx.at[slot]], dnb.at[g], sem_d)

    def _b_wait_gathers(g, slot):
        pltpu.make_async_copy(nh_hbm.at[sidx.at[slot]], gb[g], sem_g).wait()
        pltpu.make_async_copy(den_sh.at[sidx.at[slot]], dnb.at[g], sem_d).wait()

    with jax.named_scope("ph_b"):
        pltpu.sync_copy(ei_hbm.at[0, row0], sidx.at[0])
        pltpu.sync_copy(ei_hbm.at[1, row0], didx.at[0])
        pltpu.sync_copy(bw_hbm.at[row0], bwb.at[0])
        _b_gathers(0, 0, 0)
        pltpu.async_copy(ei_hbm.at[0, row0 + 1], sidx.at[1], sem_r)
        pltpu.async_copy(ei_hbm.at[1, row0 + 1], didx.at[1], sem_r)
        pltpu.async_copy(bw_hbm.at[row0 + 1], bwb.at[1], sem_r)

        @pl.loop(0, ROWS_B // 4)
        def _(t):
            for j in range(4):
                ch = t * 4 + j
                g = j % 2
                ng = (j + 1) % 2
                nslot = (j + 1) % 4
                n2slot = (j + 2) % 4

                @pl.when(ch + 1 < ROWS_B)
                def _():
                    pltpu.make_async_copy(ei_hbm.at[0, row0 + ch + 1], sidx.at[nslot], sem_r).wait()
                    pltpu.make_async_copy(ei_hbm.at[1, row0 + ch + 1], didx.at[nslot], sem_r).wait()
                    pltpu.make_async_copy(bw_hbm.at[row0 + ch + 1], bwb.at[nslot], sem_r).wait()

                    @pl.when(ch >= 1)
                    def _():
                        pltpu.make_async_copy(nh_hbm.at[pl.ds(0, CHUNK)], gb[ng], sem_s).wait()

                    _b_gathers(ch + 1, ng, nslot)

                @pl.when(ch + 2 < ROWS_B)
                def _():
                    pltpu.async_copy(ei_hbm.at[0, row0 + ch + 2], sidx.at[n2slot], sem_r)
                    pltpu.async_copy(ei_hbm.at[1, row0 + ch + 2], didx.at[n2slot], sem_r)
                    pltpu.async_copy(bw_hbm.at[row0 + ch + 2], bwb.at[n2slot], sem_r)

                _b_wait_gathers(g, j)

                for v in range(8):
                    sl = pl.ds(v * 16, 16)
                    pbuf[sl] = jnp.exp(bwb[j, sl] - mv) / dnb[g, sl]

                _scale_rows(gb[g], pbuf)
                pltpu.async_copy(gb[g], h_sh.at[didx.at[j]], sem_s, add=True)

        pltpu.make_async_copy(nh_hbm.at[pl.ds(0, CHUNK)], gb[0], sem_s).wait()
        pltpu.make_async_copy(nh_hbm.at[pl.ds(0, CHUNK)], gb[1], sem_s).wait()
        plsc.subcore_barrier()

    with jax.named_scope("ph_c"):
        for k in range(5):
            r0 = base + k * CHUNK
            pltpu.sync_copy(h_sh.at[pl.ds(r0, CHUNK)], gb[0])
            pltpu.sync_copy(gb[0], out_hbm.at[c, pl.ds(r0, CHUNK)])


@jax.jit
def kernel(feat, edge_index, edge_weight, beta, eps):
    ewp = jnp.pad(edge_weight.reshape(R, CHUNK), ((0, RP - R), (0, 0)))
    pad_idx = (jnp.arange((RP - R) * CHUNK, dtype=jnp.int32) % N).reshape(
        1, RP - R, CHUNK)
    ei3 = jnp.concatenate(
        [edge_index.astype(jnp.int32).reshape(2, R, CHUNK),
         jnp.broadcast_to(pad_idx, (2, RP - R, CHUNK))], axis=1)
    beta2 = beta.reshape(1, 1)
    eps2 = eps.reshape(1, 1)

    nh, bw, m = pl.pallas_call(
        _prep_body,
        grid=(5,),
        in_specs=[
            pl.BlockSpec(memory_space=pltpu.SMEM),
            pl.BlockSpec((2000, 128), lambda i: (i, 0)),
            pl.BlockSpec((512, 128), lambda i: (i, 0)),
        ],
        out_specs=[
            pl.BlockSpec((2000, 128), lambda i: (i, 0)),
            pl.BlockSpec((512, 128), lambda i: (i, 0)),
            pl.BlockSpec((8, 128), lambda i: (0, 0)),
        ],
        out_shape=[
            jax.ShapeDtypeStruct((N, D), jnp.float32),
            jax.ShapeDtypeStruct((RP, CHUNK), jnp.float32),
            jax.ShapeDtypeStruct((8, 128), jnp.float32),
        ],
    )(beta2, feat, ewp)

    sc_call = pl.kernel(
        _sc_body,
        out_type=jax.ShapeDtypeStruct((NC, NP, D), jnp.float32),
        mesh=plsc.VectorSubcoreMesh(core_axis_name="c", subcore_axis_name="s"),
        scratch_types=[
            pltpu.VMEM_SHARED((NP, D), jnp.float32),
            pltpu.VMEM_SHARED((NP,), jnp.float32),
            pltpu.VMEM((4, CHUNK), jnp.int32),
            pltpu.VMEM((4, CHUNK), jnp.int32),
            pltpu.VMEM((4, CHUNK), jnp.float32),
            pltpu.VMEM((8, CHUNK), jnp.int32),
            pltpu.VMEM((8, CHUNK), jnp.float32),
            pltpu.VMEM((2, CHUNK, CHUNK), jnp.float32),
            pltpu.VMEM((CHUNK,), jnp.float32),
            pltpu.VMEM((2, CHUNK), jnp.float32),
            pltpu.VMEM((CHUNK,), jnp.float32),
            pltpu.VMEM((CHUNK,), jnp.float32),
            pltpu.SemaphoreType.DMA,
            pltpu.SemaphoreType.DMA,
            pltpu.SemaphoreType.DMA,
            pltpu.SemaphoreType.DMA,
        ],
    )
    hout = sc_call(nh, bw, m[0], ei3)

    out = pl.pallas_call(
        _out_body,
        grid=(5,),
        in_specs=[
            pl.BlockSpec(memory_space=pltpu.SMEM),
            pl.BlockSpec((2000, 128), lambda i: (i, 0)),
            pl.BlockSpec((1, 2000, 128), lambda i: (0, i, 0)),
            pl.BlockSpec((1, 2000, 128), lambda i: (1, i, 0)),
        ],
        out_specs=pl.BlockSpec((2000, 128), lambda i: (i, 0)),
        out_shape=jax.ShapeDtypeStruct((N, D), jnp.float32),
    )(eps2, feat, hout, hout)

    return out
